# baseline (speedup 1.0000x reference)
import functools
import math

import jax
import jax.numpy as jnp
from jax import lax
from jax.experimental import pallas as pl
from jax.experimental.pallas import tpu as pltpu
from jax.experimental.pallas import tpu_sc as plsc

D_MODEL = 128
SCALE = math.sqrt(D_MODEL)
NUM_CORES = 2
NUM_SUBCORES = 16
NUM_WORKERS = NUM_CORES * NUM_SUBCORES
LANES = 16
UNROLL = 4
S = 320
NBUF = 2


def _sc_embed(x_flat, table):
    num_rows = table.shape[0]
    B = x_flat.shape[0]
    assert B % (NUM_WORKERS * S) == 0
    b_per_w = B // NUM_WORKERS
    n = b_per_w // S
    N = NBUF
    assert n % N == 0 and n > N
    mesh = plsc.VectorSubcoreMesh(core_axis_name="c", subcore_axis_name="s")

    @functools.partial(
        pl.kernel,
        mesh=mesh,
        out_type=jax.ShapeDtypeStruct((B * D_MODEL,), jnp.float32),
        compiler_params=pltpu.CompilerParams(needs_layout_passes=False),
        scratch_types=[pltpu.VMEM((num_rows * D_MODEL,), jnp.float32)]
        + [pltpu.VMEM((9 * 2 * D_MODEL,), jnp.float32)]
        + [pltpu.VMEM((S,), jnp.int32)] * N
        + [pltpu.VMEM((S * D_MODEL,), jnp.float32)] * N
        + [pltpu.SemaphoreType.DMA] * (2 * N),
    )
    def k(x_hbm, tbl_hbm, out_hbm, tv, tp, *rest):
        idx_v, rows_v = rest[0:N], rest[N:2 * N]
        sem_i, sem_w = rest[2 * N:3 * N], rest[3 * N:4 * N]
        wid = lax.axis_index("s") * NUM_CORES + lax.axis_index("c")
        base0 = wid * b_per_w

        pltpu.sync_copy(tbl_hbm, tv)
        for j in range(num_rows * D_MODEL // LANES):
            sl = pl.ds(j * LANES, LANES)
            tv[sl] = tv[sl] * SCALE

        def build_p(p, carry):
            digs = [p // 3, p % 3]
            for kk in range(2):
                bgq = digs[kk] * D_MODEL
                bsq = p * (2 * D_MODEL) + kk * D_MODEL
                for j in range(D_MODEL // LANES):
                    tp[pl.ds(bsq + j * LANES, LANES)] = (
                        tv[pl.ds(bgq + j * LANES, LANES)])
            return carry

        lax.fori_loop(0, 9, build_p, 0)

        lane2 = lax.iota(jnp.int32, LANES) * 2

        def idx_start(chunk, b):
            start = jnp.minimum(base0 + chunk * S, B - S)
            pltpu.async_copy(x_hbm.at[pl.ds(start, S)], idx_v[b], sem_i[b])

        def idx_wait(b):
            pltpu.make_async_copy(
                x_hbm.at[pl.ds(0, S)], idx_v[b], sem_i[b]).wait()

        def write_start(chunk, b):
            start = (base0 + chunk * S) * D_MODEL
            pltpu.async_copy(
                rows_v[b], out_hbm.at[pl.ds(start, S * D_MODEL)], sem_w[b])

        def write_wait(b):
            pltpu.make_async_copy(
                rows_v[b], out_hbm.at[pl.ds(0, S * D_MODEL)], sem_w[b]).wait()

        def compute_chunk(b):
            @plsc.parallel_loop(0, S // (2 * LANES), 1, unroll=2)
            def c_body(blk):
                c0 = blk * (2 * LANES)
                li = lane2 + c0
                g0 = plsc.load_gather(idx_v[b], [li])
                g1 = plsc.load_gather(idx_v[b], [li + 1])
                pv = (g0 * 3 + g1) * (2 * D_MODEL)
                for u in range(LANES):
                    bg = pv[u]
                    bs = (c0 + 2 * u) * D_MODEL
                    for j in range(2 * D_MODEL // LANES):
                        rows_v[b][pl.ds(bs + j * LANES, LANES)] = (
                            tp[pl.ds(bg + j * LANES, LANES)])

        for b in range(N):
            idx_start(b, b)
        for g in range(N):
            idx_wait(g)
            compute_chunk(g)
            write_start(g, g)
            idx_start(g + N, g)

        def round_body(r, carry):
            g0 = r * N
            for j in range(N):
                g = g0 + j
                b = j
                idx_wait(b)
                write_wait(b)
                compute_chunk(b)
                write_start(g, b)
                idx_start(g + N, b)
            return carry

        lax.fori_loop(1, n // N, round_body, 0)

        for b in range(N):
            write_wait(b)
            idx_wait(b)

    return k(x_flat, table.reshape(num_rows * D_MODEL))


def kernel(x, table):
    nb, t = x.shape
    out = _sc_embed(x.reshape(nb * t), table)
    return out.reshape(nb, t, D_MODEL)


# speedup vs baseline: 2.5320x; 2.5320x over previous
import functools
import math

import jax
import jax.numpy as jnp
from jax import lax
from jax.experimental import pallas as pl
from jax.experimental.pallas import tpu as pltpu
from jax.experimental.pallas import tpu_sc as plsc

D_MODEL = 128
SCALE = math.sqrt(D_MODEL)
NUM_CORES = 2
NUM_SUBCORES = 16
NUM_WORKERS = NUM_CORES * NUM_SUBCORES
LANES = 16
UNROLL = 4
S = 320
NBUF = 2


def _sc_embed(x_flat, table):
    num_rows = table.shape[0]
    B = x_flat.shape[0]
    assert B % (NUM_WORKERS * S) == 0
    b_per_w = B // NUM_WORKERS
    n = b_per_w // S
    N = NBUF
    assert n % N == 0 and n > N
    mesh = plsc.VectorSubcoreMesh(core_axis_name="c", subcore_axis_name="s")

    @functools.partial(
        pl.kernel,
        mesh=mesh,
        out_type=jax.ShapeDtypeStruct((B * D_MODEL,), jnp.float32),
        compiler_params=pltpu.CompilerParams(needs_layout_passes=False),
        scratch_types=[pltpu.VMEM((num_rows * D_MODEL,), jnp.float32)]
        + [pltpu.VMEM((S,), jnp.int32)] * N
        + [pltpu.VMEM((S * D_MODEL,), jnp.float32)] * N
        + [pltpu.SemaphoreType.DMA] * (2 * N),
    )
    def k(x_hbm, tbl_hbm, out_hbm, tv, *rest):
        idx_v, rows_v = rest[0:N], rest[N:2 * N]
        sem_i, sem_w = rest[2 * N:3 * N], rest[3 * N:4 * N]
        wid = lax.axis_index("s") * NUM_CORES + lax.axis_index("c")
        base0 = wid * b_per_w

        pltpu.sync_copy(tbl_hbm, tv)
        for j in range(num_rows * D_MODEL // LANES):
            sl = pl.ds(j * LANES, LANES)
            tv[sl] = tv[sl] * SCALE

        lane_off = lax.iota(jnp.int32, LANES) * D_MODEL

        def idx_start(chunk, b):
            start = jnp.minimum(base0 + chunk * S, B - S)
            pltpu.async_copy(x_hbm.at[pl.ds(start, S)], idx_v[b], sem_i[b])

        def idx_wait(b):
            pltpu.make_async_copy(
                x_hbm.at[pl.ds(0, S)], idx_v[b], sem_i[b]).wait()

        def write_start(chunk, b):
            start = (base0 + chunk * S) * D_MODEL
            pltpu.async_copy(
                rows_v[b], out_hbm.at[pl.ds(start, S * D_MODEL)], sem_w[b])

        def write_wait(b):
            pltpu.make_async_copy(
                rows_v[b], out_hbm.at[pl.ds(0, S * D_MODEL)], sem_w[b]).wait()

        def compute_chunk(b):
            @plsc.parallel_loop(0, S // LANES, 1, unroll=2)
            def c_body(blk):
                c0 = blk * LANES
                cb = idx_v[b][pl.ds(c0, LANES)]
                bgs = [cb[u] * D_MODEL for u in range(LANES)]
                for u in range(LANES):
                    bs = (c0 + u) * D_MODEL
                    for j in range(D_MODEL // LANES):
                        rows_v[b][pl.ds(bs + j * LANES, LANES)] = (
                            tv[pl.ds(bgs[u] + j * LANES, LANES)])

        for b in range(N):
            idx_start(b, b)
        for g in range(N):
            idx_wait(g)
            compute_chunk(g)
            write_start(g, g)
            idx_start(g + N, g)

        def round_body(r, carry):
            g0 = r * N
            for j in range(N):
                g = g0 + j
                b = j
                idx_wait(b)
                write_wait(b)
                compute_chunk(b)
                write_start(g, b)
                idx_start(g + N, b)
            return carry

        lax.fori_loop(1, n // N, round_body, 0)

        for b in range(N):
            write_wait(b)
            idx_wait(b)

    return k(x_flat, table.reshape(num_rows * D_MODEL))


def kernel(x, table):
    nb, t = x.shape
    out = _sc_embed(x.reshape(nb * t), table)
    return out.reshape(nb, t, D_MODEL)
